# baseline (speedup 1.0000x reference)
import functools

import jax
import jax.numpy as jnp
from jax import lax
from jax.experimental import pallas as pl
from jax.experimental.pallas import tpu as pltpu
from jax.experimental.pallas import tpu_sc as plsc

VOCAB = 1000000
EMBED = 64
B = 4096 * 200
IDXW = 128
NW = 32
ROWS_PER_W = B // NW
SUPER = 1024
CHUNK = 512
N_SUPER = ROWS_PER_W // SUPER
SCALE = 8.0

_mesh = plsc.VectorSubcoreMesh(core_axis_name="c", subcore_axis_name="s")


@functools.partial(
    pl.kernel,
    mesh=_mesh,
    out_type=jax.ShapeDtypeStruct((B, EMBED), jnp.float32),
    scratch_types=[
        pltpu.VMEM((SUPER // IDXW, IDXW), jnp.int32),
        pltpu.VMEM((CHUNK, EMBED), jnp.float32),
        pltpu.SemaphoreType.DMA,
    ],
    compiler_params=pltpu.CompilerParams(use_tc_tiling_on_sc=False),
)
def _embed(idx_hbm, table_hbm, out_hbm, idx_v, rows_v, sem):
    wid = lax.axis_index("s") * 2 + lax.axis_index("c")
    row_base = wid * ROWS_PER_W

    def super_body(si, carry):
        srow0 = pl.multiple_of(row_base + si * SUPER, SUPER)
        irow0 = pl.multiple_of(srow0 // IDXW, SUPER // IDXW)
        pltpu.sync_copy(idx_hbm.at[pl.ds(irow0, SUPER // IDXW)], idx_v)
        for h in range(SUPER // CHUNK):
            copies = []
            for j in range(CHUNK // IDXW):
                copies.append(
                    pltpu.async_copy(
                        table_hbm.at[idx_v.at[h * (CHUNK // IDXW) + j]],
                        rows_v.at[pl.ds(j * IDXW, IDXW)],
                        sem,
                    )
                )
            for c in copies:
                c.wait()

            def scale_body(i, c):
                for j in range(EMBED // 16):
                    sl = (i, pl.ds(j * 16, 16))
                    rows_v[sl] = rows_v[sl] * SCALE
                return c

            lax.fori_loop(0, CHUNK, scale_body, 0, unroll=2)
            pltpu.sync_copy(
                rows_v, out_hbm.at[pl.ds(srow0 + h * CHUNK, CHUNK)]
            )
        return carry

    lax.fori_loop(0, N_SUPER, super_body, 0)


def kernel(token_ids, table):
    ids = token_ids.astype(jnp.int32).reshape(B // IDXW, IDXW)
    out = _embed(ids, table)
    return out.reshape(token_ids.shape + (EMBED,))
